# Initial kernel scaffold; baseline (speedup 1.0000x reference)
import jax
import jax.numpy as jnp
from jax.experimental import pallas as pl


def kernel(x, embedding_weight):
    raise NotImplementedError("write your pallas kernel here")


# speedup vs baseline: 1.0773x; 1.0773x over previous
import functools

import jax
import jax.numpy as jnp
from jax import lax
from jax.experimental import pallas as pl
from jax.experimental.pallas import tpu as pltpu
from jax.experimental.pallas import tpu_sc as plsc

D = 128
CHUNK = 112
NCH = 28

_info = plsc.get_sparse_core_info()
NC = _info.num_cores
NS = _info.num_subcores
NW = NC * NS
BP = NW * NCH * CHUNK


def _make_gather(num_rows):
    mesh = plsc.VectorSubcoreMesh(core_axis_name="c", subcore_axis_name="s")

    @functools.partial(
        pl.kernel,
        mesh=mesh,
        out_type=jax.ShapeDtypeStruct((BP, D), jnp.float32),
        scratch_types=[
            pltpu.VMEM((NCH, CHUNK), jnp.int32),
            pltpu.VMEM((CHUNK, D), jnp.float32),
            pltpu.VMEM((CHUNK, D), jnp.float32),
            pltpu.SemaphoreType.DMA,
            pltpu.SemaphoreType.DMA,
        ],
    )
    def gather(idx_hbm, table_hbm, out_hbm, idx_v, rows0, rows1, sem0, sem1):
        wid = lax.axis_index("s") * NC + lax.axis_index("c")
        base = wid * (NCH * CHUNK)
        pltpu.sync_copy(idx_hbm.at[wid], idx_v)

        pltpu.async_copy(table_hbm.at[idx_v.at[0]], rows0, sem0)

        def body(g, _):
            i0 = 2 * g
            i1 = i0 + 1
            pltpu.async_copy(table_hbm.at[idx_v.at[i1]], rows1, sem1)
            pltpu.make_async_copy(table_hbm.at[idx_v.at[i0]], rows0, sem0).wait()
            pltpu.sync_copy(rows0, out_hbm.at[pl.ds(base + i0 * CHUNK, CHUNK)])

            @pl.when(g < NCH // 2 - 1)
            def _():
                pltpu.async_copy(table_hbm.at[idx_v.at[i1 + 1]], rows0, sem0)

            pltpu.make_async_copy(table_hbm.at[idx_v.at[i1]], rows1, sem1).wait()
            pltpu.sync_copy(rows1, out_hbm.at[pl.ds(base + i1 * CHUNK, CHUNK)])
            return 0

        lax.fori_loop(0, NCH // 2, body, 0)

    return gather


_gather = _make_gather(BP)


def kernel(x, embedding_weight):
    b = x.shape[0]
    xp = jnp.zeros((BP,), jnp.int32).at[:b].set(x).reshape(NW, NCH, CHUNK)
    out = _gather(xp, embedding_weight)
    return out[:b]


# speedup vs baseline: 1.9326x; 1.7939x over previous
import functools

import jax
import jax.numpy as jnp
from jax import lax
from jax.experimental import pallas as pl
from jax.experimental.pallas import tpu as pltpu
from jax.experimental.pallas import tpu_sc as plsc

D = 128
CHUNK = 128
NCH = 25
NBUF = 5

_info = plsc.get_sparse_core_info()
NC = _info.num_cores
NS = _info.num_subcores
NW = NC * NS
B = 100000
NFULL = (B - CHUNK) // CHUNK + 1
LAST = B - CHUNK


def _make_gather():
    mesh = plsc.VectorSubcoreMesh(core_axis_name="c", subcore_axis_name="s")

    @functools.partial(
        pl.kernel,
        mesh=mesh,
        out_type=jax.ShapeDtypeStruct((B, D), jnp.float32),
        scratch_types=[
            pltpu.VMEM((NCH, CHUNK), jnp.int32),
            pltpu.VMEM((NBUF, CHUNK, D), jnp.float32),
            pltpu.SemaphoreType.DMA((NBUF,)),
            pltpu.SemaphoreType.DMA((NBUF,)),
        ],
    )
    def gather(idx_hbm, table_hbm, out_hbm, idx_v, rows_v, gsem, ssem):
        wid = lax.axis_index("s") * NC + lax.axis_index("c")
        pltpu.sync_copy(idx_hbm.at[wid], idx_v)

        def start_gather(b, i):
            pltpu.async_copy(table_hbm.at[idx_v.at[i]], rows_v.at[b], gsem.at[b])

        def wait_gather(b):
            pltpu.make_async_copy(
                table_hbm.at[idx_v.at[0]], rows_v.at[b], gsem.at[b]).wait()

        def start_store(b, i):
            off = jnp.minimum((wid * NCH + i) * CHUNK, LAST)
            pltpu.async_copy(
                rows_v.at[b], out_hbm.at[pl.ds(off, CHUNK)], ssem.at[b])

        def wait_store(b):
            pltpu.make_async_copy(
                rows_v.at[b], out_hbm.at[pl.ds(0, CHUNK)], ssem.at[b]).wait()

        for b in range(NBUF):
            start_gather(b, b)

        def body(k, _):
            for b in range(NBUF):
                wait_gather(b)
                start_store(b, k * NBUF + b)

            @pl.when(k < NCH // NBUF - 1)
            def _():
                for b in range(NBUF):
                    wait_store(b)
                    start_gather(b, (k + 1) * NBUF + b)

            return 0

        lax.fori_loop(0, NCH // NBUF, body, 0)

        for b in range(NBUF):
            wait_store(b)

    return gather


_gather = _make_gather()


def kernel(x, embedding_weight):
    ntail = NW * NCH - NFULL
    xg = jnp.concatenate([x[:NFULL * CHUNK],
                          jnp.tile(x[LAST:], ntail)]).reshape(NW, NCH, CHUNK)
    return _gather(xg, embedding_weight)
